# baseline (speedup 1.0000x reference)
import functools

import jax
import jax.numpy as jnp
from jax import lax
from jax.experimental import pallas as pl
from jax.experimental.pallas import tpu as pltpu
from jax.experimental.pallas import tpu_sc as plsc

B, ORDERS, H, F = 4096, 3, 16, 20
EMBED = 128
BAGS = B * ORDERS
RPB = H * F
NCH, CH = 4, 80
NC, NS = 2, 16
NW = NC * NS
PER_W = BAGS // NW
GROUP = 16
SCALE = 1.0 / F


def _pool_body(idx_hbm, w_hbm, table_hbm, out_hbm,
               idx0, idx1, w0, w1, rows0, rows1, res_v,
               isem0, isem1, rsem0, rsem1):
  wid = lax.axis_index("s") * NC + lax.axis_index("c")
  base = wid * PER_W
  last = base + PER_W - 1

  def issue_gathers(idx_buf, rows_buf, sem):
    for c in range(NCH):
      pltpu.async_copy(table_hbm.at[idx_buf.at[0, c]],
                       rows_buf.at[pl.ds(c * CH, CH)], sem)

  def drain_gathers(rows_buf, sem):
    for c in range(NCH):
      pltpu.make_async_copy(table_hbm.at[pl.ds(0, CH)],
                            rows_buf.at[pl.ds(c * CH, CH)], sem).wait()

  def issue_prefetch(row, idx_buf, w_buf, sem):
    pltpu.async_copy(idx_hbm.at[pl.ds(row, 1)], idx_buf, sem)
    pltpu.async_copy(w_hbm.at[pl.ds(row, 1)], w_buf, sem)

  def drain_prefetch(idx_buf, w_buf, sem):
    pltpu.make_async_copy(idx_hbm.at[pl.ds(0, 1)], idx_buf, sem).wait()
    pltpu.make_async_copy(w_hbm.at[pl.ds(0, 1)], w_buf, sem).wait()

  def compute(rows_buf, w_buf, jj):
    def rbody(ti, a):
      r0 = ti * 16
      c = ti // (CH // 16)
      off = (ti % (CH // 16)) * 16
      wv = w_buf[0, c, pl.ds(off, 16)]
      a = list(a)
      for jl in range(16):
        w = wv[jl]
        for d in range(8):
          a[d] = a[d] + w * rows_buf[r0 + jl, pl.ds(d * 16, 16)]
      return tuple(a)

    acc = lax.fori_loop(
        0, RPB // 16, rbody,
        tuple(jnp.zeros((16,), jnp.float32) for _ in range(8)))
    for d in range(8):
      res_v[jj, pl.ds(d * 16, 16)] = acc[d] * SCALE

  pltpu.sync_copy(idx_hbm.at[pl.ds(base, 1)], idx0)
  pltpu.sync_copy(w_hbm.at[pl.ds(base, 1)], w0)
  issue_gathers(idx0, rows0, rsem0)
  issue_prefetch(base + 1, idx1, w1, isem1)

  def body(t, carry):
    j0 = 2 * t
    j1 = 2 * t + 1
    drain_prefetch(idx1, w1, isem1)
    issue_gathers(idx1, rows1, rsem1)
    drain_gathers(rows0, rsem0)
    compute(rows0, w0, j0 % GROUP)
    issue_prefetch(jnp.minimum(base + j0 + 2, last), idx0, w0, isem0)
    drain_prefetch(idx0, w0, isem0)
    issue_gathers(idx0, rows0, rsem0)
    drain_gathers(rows1, rsem1)
    compute(rows1, w1, j1 % GROUP)

    @pl.when(t % (GROUP // 2) == (GROUP // 2 - 1))
    def _flush():
      pltpu.sync_copy(res_v,
                      out_hbm.at[pl.ds(base + (t // (GROUP // 2)) * GROUP,
                                       GROUP)])

    issue_prefetch(jnp.minimum(base + j1 + 2, last), idx1, w1, isem1)
    return carry

  lax.fori_loop(0, PER_W // 2, body, 0)
  drain_prefetch(idx1, w1, isem1)
  drain_gathers(rows0, rsem0)


_pool = functools.partial(
    pl.kernel,
    out_type=jax.ShapeDtypeStruct((BAGS, EMBED), jnp.float32),
    mesh=plsc.VectorSubcoreMesh(
        core_axis_name="c", subcore_axis_name="s", num_cores=NC,
        num_subcores=NS),
    scratch_types=[
        pltpu.VMEM((1, NCH, CH), jnp.int32),
        pltpu.VMEM((1, NCH, CH), jnp.int32),
        pltpu.VMEM((1, NCH, CH), jnp.float32),
        pltpu.VMEM((1, NCH, CH), jnp.float32),
        pltpu.VMEM((RPB, EMBED), jnp.float32),
        pltpu.VMEM((RPB, EMBED), jnp.float32),
        pltpu.VMEM((GROUP, EMBED), jnp.float32),
        pltpu.SemaphoreType.DMA,
        pltpu.SemaphoreType.DMA,
        pltpu.SemaphoreType.DMA,
        pltpu.SemaphoreType.DMA,
    ],
)(_pool_body)


def _mlp_body(x_ref, w1_ref, b1_ref, w2_ref, b2_ref, out_ref):
  m = lax.dot_general(w1_ref[...], w2_ref[...], (((0,), (1,)), ((), ())),
                      preferred_element_type=jnp.float32)
  bias = lax.dot_general(b1_ref[...], w2_ref[...], (((1,), (1,)), ((), ())),
                         preferred_element_type=jnp.float32)
  bias = bias + b2_ref[...]
  logits = lax.dot_general(x_ref[...], m, (((1,), (0,)), ((), ())),
                           preferred_element_type=jnp.float32)
  logits = logits + bias
  mx = jnp.max(logits, axis=-1, keepdims=True)
  lse = jnp.log(jnp.sum(jnp.exp(logits - mx), axis=-1, keepdims=True)) + mx
  out_ref[...] = logits - lse


_mlp = pl.pallas_call(
    _mlp_body,
    out_shape=jax.ShapeDtypeStruct((B, 128), jnp.float32),
)


def kernel(ngrams, ngrams_weights, emb_table, W1, b1, W2, b2):
  idx = ngrams.reshape(BAGS, NCH, CH)
  w = ngrams_weights.reshape(BAGS, NCH, CH)
  pooled = _pool(idx, w, emb_table)
  embed = pooled.reshape(B, ORDERS * EMBED)
  return _mlp(embed, W1, b1.reshape(1, -1), W2, b2.reshape(1, -1))


# speedup vs baseline: 1.0089x; 1.0089x over previous
import functools

import jax
import jax.numpy as jnp
from jax import lax
from jax.experimental import pallas as pl
from jax.experimental.pallas import tpu as pltpu
from jax.experimental.pallas import tpu_sc as plsc

B, ORDERS, H, F = 4096, 3, 16, 20
EMBED = 128
BAGS = B * ORDERS
RPB = H * F
NC, NS = 2, 16
NW = NC * NS
PER_W = BAGS // NW
CHUNKS = ((0, 128), (128, 128), (256, 64))
GROUP = 16
SCALE = 1.0 / F


def _pool_body(idx_hbm, w_hbm, table_hbm, out_hbm,
               idx0, idx1, w0, w1, rows0, rows1, res_v,
               isem0, isem1, rsem0, rsem1):
  wid = lax.axis_index("s") * NC + lax.axis_index("c")
  base = wid * PER_W
  last = base + PER_W - 1

  def issue_gathers(idx_buf, rows_buf, sem):
    for off, n in CHUNKS:
      pltpu.async_copy(table_hbm.at[idx_buf.at[0, pl.ds(off, n)]],
                       rows_buf.at[pl.ds(off, n)], sem)

  def drain_gathers(rows_buf, sem):
    for off, n in CHUNKS:
      pltpu.make_async_copy(table_hbm.at[pl.ds(0, n)],
                            rows_buf.at[pl.ds(off, n)], sem).wait()

  def issue_prefetch(row, idx_buf, w_buf, sem):
    pltpu.async_copy(idx_hbm.at[pl.ds(row, 1)], idx_buf, sem)
    pltpu.async_copy(w_hbm.at[pl.ds(row, 1)], w_buf, sem)

  def drain_prefetch(idx_buf, w_buf, sem):
    pltpu.make_async_copy(idx_hbm.at[pl.ds(0, 1)], idx_buf, sem).wait()
    pltpu.make_async_copy(w_hbm.at[pl.ds(0, 1)], w_buf, sem).wait()

  def compute(rows_buf, w_buf, jj):
    def rbody(ti, a):
      r0 = ti * 16
      wv = w_buf[0, pl.ds(r0, 16)]
      a = list(a)
      for jl in range(16):
        w = wv[jl]
        for d in range(8):
          a[d] = a[d] + w * rows_buf[r0 + jl, pl.ds(d * 16, 16)]
      return tuple(a)

    acc = lax.fori_loop(
        0, RPB // 16, rbody,
        tuple(jnp.zeros((16,), jnp.float32) for _ in range(8)))
    for d in range(8):
      res_v[jj, pl.ds(d * 16, 16)] = acc[d] * SCALE

  pltpu.sync_copy(idx_hbm.at[pl.ds(base, 1)], idx0)
  pltpu.sync_copy(w_hbm.at[pl.ds(base, 1)], w0)
  issue_gathers(idx0, rows0, rsem0)
  issue_prefetch(base + 1, idx1, w1, isem1)

  def body(t, carry):
    j0 = 2 * t
    j1 = 2 * t + 1
    drain_prefetch(idx1, w1, isem1)
    issue_gathers(idx1, rows1, rsem1)
    drain_gathers(rows0, rsem0)
    compute(rows0, w0, j0 % GROUP)
    issue_prefetch(jnp.minimum(base + j0 + 2, last), idx0, w0, isem0)
    drain_prefetch(idx0, w0, isem0)
    issue_gathers(idx0, rows0, rsem0)
    drain_gathers(rows1, rsem1)
    compute(rows1, w1, j1 % GROUP)

    @pl.when(t % (GROUP // 2) == (GROUP // 2 - 1))
    def _flush():
      pltpu.sync_copy(res_v,
                      out_hbm.at[pl.ds(base + (t // (GROUP // 2)) * GROUP,
                                       GROUP)])

    issue_prefetch(jnp.minimum(base + j1 + 2, last), idx1, w1, isem1)
    return carry

  lax.fori_loop(0, PER_W // 2, body, 0)
  drain_prefetch(idx1, w1, isem1)
  drain_gathers(rows0, rsem0)


_pool = functools.partial(
    pl.kernel,
    out_type=jax.ShapeDtypeStruct((BAGS, EMBED), jnp.float32),
    mesh=plsc.VectorSubcoreMesh(
        core_axis_name="c", subcore_axis_name="s", num_cores=NC,
        num_subcores=NS),
    scratch_types=[
        pltpu.VMEM((1, RPB), jnp.int32),
        pltpu.VMEM((1, RPB), jnp.int32),
        pltpu.VMEM((1, RPB), jnp.float32),
        pltpu.VMEM((1, RPB), jnp.float32),
        pltpu.VMEM((RPB, EMBED), jnp.float32),
        pltpu.VMEM((RPB, EMBED), jnp.float32),
        pltpu.VMEM((GROUP, EMBED), jnp.float32),
        pltpu.SemaphoreType.DMA,
        pltpu.SemaphoreType.DMA,
        pltpu.SemaphoreType.DMA,
        pltpu.SemaphoreType.DMA,
    ],
)(_pool_body)


def _mlp_body(x_ref, w1_ref, b1_ref, w2_ref, b2_ref, out_ref):
  m = lax.dot_general(w1_ref[...], w2_ref[...], (((0,), (1,)), ((), ())),
                      preferred_element_type=jnp.float32)
  bias = lax.dot_general(b1_ref[...], w2_ref[...], (((1,), (1,)), ((), ())),
                         preferred_element_type=jnp.float32)
  bias = bias + b2_ref[...]
  logits = lax.dot_general(x_ref[...], m, (((1,), (0,)), ((), ())),
                           preferred_element_type=jnp.float32)
  logits = logits + bias
  mx = jnp.max(logits, axis=-1, keepdims=True)
  lse = jnp.log(jnp.sum(jnp.exp(logits - mx), axis=-1, keepdims=True)) + mx
  out_ref[...] = logits - lse


_mlp = pl.pallas_call(
    _mlp_body,
    out_shape=jax.ShapeDtypeStruct((B, 128), jnp.float32),
)


def kernel(ngrams, ngrams_weights, emb_table, W1, b1, W2, b2):
  idx = ngrams.reshape(BAGS, RPB)
  w = ngrams_weights.reshape(BAGS, RPB)
  pooled = _pool(idx, w, emb_table)
  embed = pooled.reshape(B, ORDERS * EMBED)
  return _mlp(embed, W1, b1.reshape(1, -1), W2, b2.reshape(1, -1))
